# baseline (speedup 1.0000x reference)
import functools

import jax
import jax.numpy as jnp
from jax import lax
from jax.experimental import pallas as pl
from jax.experimental.pallas import tpu as pltpu
from jax.experimental.pallas import tpu_sc as plsc

VOCAB = 1000000
SEQ = 200
DIM = 64
BATCH = 4096
NROWS = BATCH * SEQ
NC, NS, LANES = 2, 16, 16
NW = NC * NS
BPW = NROWS // NW
C = 800
NIT = BPW // C
GC = 80
NG = C // GC


def _emb_body(idx_hbm, tok_hbm, pos_hbm, out_hbm, idx_v, rows_v, pos_v, sem):
    wid = lax.axis_index("s") * NC + lax.axis_index("c")
    base = wid * BPW
    pltpu.sync_copy(pos_hbm, pos_v)

    def chunk_body(i, carry):
        off = base + i * C
        pltpu.sync_copy(idx_hbm.at[pl.ds(off, C)], idx_v)
        copies = [
            pltpu.async_copy(
                tok_hbm.at[idx_v.at[pl.ds(g * GC, GC)]],
                rows_v.at[pl.ds(g * GC, GC)],
                sem,
            )
            for g in range(NG)
        ]
        for cp in copies:
            cp.wait()

        def add_body(r, carry2):
            for rep in range(C // SEQ):
                for j in range(DIM // LANES):
                    sl = pl.ds(j * LANES, LANES)
                    rows_v[rep * SEQ + r, sl] = (
                        rows_v[rep * SEQ + r, sl] + pos_v[r, sl]
                    )
            return carry2

        lax.fori_loop(0, SEQ, add_body, 0, unroll=2)
        pltpu.sync_copy(rows_v, out_hbm.at[pl.ds(off, C)])
        return carry

    lax.fori_loop(0, NIT, chunk_body, 0)


@functools.partial(jax.jit, static_argnames=())
def kernel(inputs, token_table, pos_table):
    idx = inputs.reshape(-1).astype(jnp.int32)
    mesh = plsc.VectorSubcoreMesh(core_axis_name="c", subcore_axis_name="s")
    run = pl.kernel(
        _emb_body,
        out_type=jax.ShapeDtypeStruct((NROWS, DIM), jnp.float32),
        mesh=mesh,
        scratch_types=[
            pltpu.VMEM((C,), jnp.int32),
            pltpu.VMEM((C, DIM), jnp.float32),
            pltpu.VMEM((SEQ, DIM), jnp.float32),
            pltpu.SemaphoreType.DMA,
        ],
        compiler_params=pltpu.CompilerParams(use_tc_tiling_on_sc=False),
    )
    out = run(idx, token_table, pos_table)
    return out.reshape(BATCH, SEQ, DIM)
